# baseline (speedup 1.0000x reference)
import jax
import jax.numpy as jnp
from jax import lax
from jax.experimental import pallas as pl
from jax.experimental.pallas import tpu as pltpu
from jax.experimental.pallas import tpu_sc as plsc

N = 100000
E = 3200000
C = 16

NC = 2
NS = 16
NW = NC * NS

GSZ = 768
G_FULL = E // GSZ
G_REM = E - G_FULL * GSZ
BASE_G = G_FULL // NW
EXTRA = G_FULL % NW

ACC_ROWS = 100016
ZROWS = ACC_ROWS // NS
OROWS = N // NS


def _hop_body(x_hbm, src_hbm, dst_hbm, pads_hbm, zeros_hbm, part_hbm,
              sbuf, dbuf, rows, acc, isem, gsem, ssem):
    c = lax.axis_index("c")
    s = lax.axis_index("s")
    wid = c * NS + s
    g0 = BASE_G * wid + jnp.minimum(wid, EXTRA)
    ng = BASE_G + jnp.where(wid < EXTRA, 1, 0)

    pltpu.sync_copy(zeros_hbm, acc.at[pl.ds(s * ZROWS, ZROWS)])
    plsc.subcore_barrier()

    def fire_idx(g):
        r = g % 4
        q = (g0 + g) * GSZ
        pltpu.async_copy(src_hbm.at[pl.ds(q, GSZ)], sbuf.at[r], isem.at[r])
        pltpu.async_copy(dst_hbm.at[pl.ds(q, GSZ)], dbuf.at[r], isem.at[r])

    def wait_idx(g):
        r = g % 4
        q = (g0 + g) * GSZ
        pltpu.make_async_copy(
            src_hbm.at[pl.ds(q, GSZ)], sbuf.at[r], isem.at[r]).wait()
        pltpu.make_async_copy(
            dst_hbm.at[pl.ds(q, GSZ)], dbuf.at[r], isem.at[r]).wait()

    def fire_gather(g):
        r, p = g % 4, g % 2
        pltpu.async_copy(x_hbm.at[sbuf.at[r]], rows.at[p], gsem.at[p])

    def wait_gather(g):
        r, p = g % 4, g % 2
        pltpu.make_async_copy(
            x_hbm.at[sbuf.at[r]], rows.at[p], gsem.at[p]).wait()

    def fire_scatter(g):
        r, p = g % 4, g % 2
        pltpu.async_copy(rows.at[p], acc.at[dbuf.at[r]], ssem, add=True)

    def wait_scatter(g):
        r, p = g % 4, g % 2
        pltpu.make_async_copy(
            rows.at[p], acc.at[dbuf.at[r]], ssem).wait()

    fire_idx(0)
    fire_idx(1)
    fire_idx(2)
    wait_idx(0)
    fire_gather(0)

    def grp_body(g, carry):
        @pl.when(g > 0)
        def _():
            wait_scatter(g - 1)

        @pl.when(g + 1 < ng)
        def _():
            wait_idx(g + 1)
            fire_gather(g + 1)

        @pl.when(g + 3 < ng)
        def _():
            fire_idx(g + 3)

        wait_gather(g)
        fire_scatter(g)
        return carry

    lax.fori_loop(0, ng, grp_body, 0)
    wait_scatter(ng - 1)

    @pl.when(wid == NW - 1)
    def _():
        pltpu.sync_copy(pads_hbm.at[1], sbuf.at[0])
        pltpu.sync_copy(pads_hbm.at[0], dbuf.at[0])
        pltpu.async_copy(x_hbm.at[sbuf.at[0]], rows.at[0], gsem.at[0])
        pltpu.make_async_copy(
            x_hbm.at[sbuf.at[0]], rows.at[0], gsem.at[0]).wait()
        pltpu.sync_copy(rows.at[0], acc.at[dbuf.at[0]], add=True)

    plsc.subcore_barrier()

    pltpu.sync_copy(acc.at[pl.ds(s * OROWS, OROWS)],
                    part_hbm.at[c, pl.ds(s * OROWS, OROWS)])


def _make_hop():
    mesh = plsc.VectorSubcoreMesh(core_axis_name="c", subcore_axis_name="s",
                                  num_cores=NC, num_subcores=NS)
    return pl.kernel(
        _hop_body,
        out_type=jax.ShapeDtypeStruct((NC, N, C), jnp.float32),
        mesh=mesh,
        scratch_types=[
            pltpu.VMEM((4, GSZ), jnp.int32),
            pltpu.VMEM((4, GSZ), jnp.int32),
            pltpu.VMEM((2, GSZ, C), jnp.float32),
            pltpu.VMEM_SHARED((ACC_ROWS, C), jnp.float32),
            pltpu.SemaphoreType.DMA((4,)),
            pltpu.SemaphoreType.DMA((2,)),
            pltpu.SemaphoreType.DMA,
        ],
        compiler_params=pltpu.CompilerParams(use_tc_tiling_on_sc=False),
    )


_CROWS = 625
_CCHUNKS = N // NW // _CROWS


def _comb_body(p_hbm, x_hbm, b0, b1, bo):
    c = lax.axis_index("c")
    s = lax.axis_index("s")
    wid = c * NS + s

    def chunk_body(ch, carry):
        base = wid * (N // NW) + ch * _CROWS
        pltpu.sync_copy(p_hbm.at[0, pl.ds(base, _CROWS)], b0)
        pltpu.sync_copy(p_hbm.at[1, pl.ds(base, _CROWS)], b1)

        def row_body(i, carry2):
            bo[i, :] = b0[i, :] + b1[i, :]
            return carry2

        lax.fori_loop(0, _CROWS, row_body, 0)
        pltpu.sync_copy(bo, x_hbm.at[pl.ds(base, _CROWS)])
        return carry

    lax.fori_loop(0, _CCHUNKS, chunk_body, 0)


def _make_combine():
    mesh = plsc.VectorSubcoreMesh(core_axis_name="c", subcore_axis_name="s",
                                  num_cores=NC, num_subcores=NS)
    return pl.kernel(
        _comb_body,
        out_type=jax.ShapeDtypeStruct((N, C), jnp.float32),
        mesh=mesh,
        scratch_types=[
            pltpu.VMEM((_CROWS, C), jnp.float32),
            pltpu.VMEM((_CROWS, C), jnp.float32),
            pltpu.VMEM((_CROWS, C), jnp.float32),
        ],
        compiler_params=pltpu.CompilerParams(use_tc_tiling_on_sc=False),
    )


_FROWS = N * C // 128
_FB = 1000


def _final_body(p_ref, wbd_ref, sb_ref, bt_ref, o_ref):
    v = p_ref[0] + p_ref[1]
    z = jnp.dot(v, wbd_ref[...], preferred_element_type=jnp.float32)
    z = z + bt_ref[...]
    groups = []
    for k in range(8):
        mk = jnp.max(z[:, 16 * k:16 * (k + 1)], axis=1, keepdims=True)
        groups.append(jnp.broadcast_to(mk, (_FB, C)))
    m = jnp.concatenate(groups, axis=1)
    zz = z - m
    e = jnp.exp(zz)
    s = jnp.dot(e, sb_ref[...], preferred_element_type=jnp.float32)
    o_ref[...] = zz - jnp.log(s)


def _final(parts, wt, b2):
    pf = parts.reshape(NC, _FROWS, 128)
    wbd = jnp.kron(jnp.eye(8, dtype=jnp.float32), wt)
    sb = jnp.kron(jnp.eye(8, dtype=jnp.float32),
                  jnp.ones((C, C), jnp.float32))
    bt = jnp.tile(b2, (1, 8))
    out = pl.pallas_call(
        _final_body,
        grid=(pl.cdiv(_FROWS, _FB),),
        in_specs=[
            pl.BlockSpec((NC, _FB, 128), lambda i: (0, i, 0)),
            pl.BlockSpec((128, 128), lambda i: (0, 0)),
            pl.BlockSpec((128, 128), lambda i: (0, 0)),
            pl.BlockSpec((1, 128), lambda i: (0, 0)),
        ],
        out_specs=pl.BlockSpec((_FB, 128), lambda i: (i, 0)),
        out_shape=jax.ShapeDtypeStruct((_FROWS, 128), jnp.float32),
    )(pf, wbd, sb, bt)
    return out.reshape(N, C)


@jax.jit
def kernel(y, adj_t, W, b):
    dst1 = adj_t[0].astype(jnp.int32)
    src1 = adj_t[1].astype(jnp.int32)
    npad = GSZ - G_REM
    tail_base = G_FULL * GSZ
    pad_dst = N + (jnp.arange(npad, dtype=jnp.int32) % (ACC_ROWS - N))
    pad_src = jnp.zeros((npad,), jnp.int32)
    pads = jnp.stack([
        jnp.concatenate([dst1[tail_base:], pad_dst]),
        jnp.concatenate([src1[tail_base:], pad_src]),
    ])
    zeros = jnp.zeros((ZROWS, C), jnp.float32)

    hop = _make_hop()
    combine = _make_combine()
    p1 = hop(y, src1, dst1, pads, zeros)
    x1 = combine(p1)
    p2 = hop(x1, src1, dst1, pads, zeros)
    return _final(p2, W.T, b.reshape(1, C))


# speedup vs baseline: 1.0544x; 1.0544x over previous
import jax
import jax.numpy as jnp
from jax import lax
from jax.experimental import pallas as pl
from jax.experimental.pallas import tpu as pltpu
from jax.experimental.pallas import tpu_sc as plsc

N = 100000
E = 3200000
C = 16

NC = 2
NS = 16
NW = NC * NS

CHUNK = 128
KB = 6
TOTCH = E // CHUNK
G_FULL = TOTCH // KB
G_REM = TOTCH - G_FULL * KB
BASE_G = G_FULL // NW
EXTRA = G_FULL % NW

AR = 100096
ZROWS = AR // NS


def _hop_body(x_hbm, adj3_hbm, pads_hbm, zeros_hbm, part_hbm,
              sbuf, dbuf, rows, acc, isem, gsem, ssem):
    c = lax.axis_index("c")
    s = lax.axis_index("s")
    wid = c * NS + s
    g0 = BASE_G * wid + jnp.minimum(wid, EXTRA)
    ng = BASE_G + jnp.where(wid < EXTRA, 1, 0)

    def fire_idx(g):
        r = g % 4
        q = (g0 + g) * KB
        pltpu.async_copy(adj3_hbm.at[1, pl.ds(q, KB)], sbuf.at[r], isem.at[r])
        pltpu.async_copy(adj3_hbm.at[0, pl.ds(q, KB)], dbuf.at[r], isem.at[r])

    def wait_idx(g):
        r = g % 4
        q = (g0 + g) * KB
        pltpu.make_async_copy(
            adj3_hbm.at[1, pl.ds(q, KB)], sbuf.at[r], isem.at[r]).wait()
        pltpu.make_async_copy(
            adj3_hbm.at[0, pl.ds(q, KB)], dbuf.at[r], isem.at[r]).wait()

    def fire_gathers(g):
        r, p = g % 4, g % 2
        for j in range(KB):
            pltpu.async_copy(x_hbm.at[sbuf.at[r, j]], rows.at[p, j],
                             gsem.at[p])

    def wait_gathers(g):
        r, p = g % 4, g % 2
        for j in range(KB):
            pltpu.make_async_copy(
                x_hbm.at[sbuf.at[r, j]], rows.at[p, j], gsem.at[p]).wait()

    def fire_scatters(g):
        r, p = g % 4, g % 2
        for j in range(KB):
            pltpu.async_copy(rows.at[p, j], acc.at[dbuf.at[r, j]], ssem,
                             add=True)

    def wait_scatters(g):
        r, p = g % 4, g % 2
        for j in range(KB):
            pltpu.make_async_copy(
                rows.at[p, j], acc.at[dbuf.at[r, j]], ssem).wait()

    fire_idx(0)
    fire_idx(1)
    fire_idx(2)
    wait_idx(0)
    fire_gathers(0)
    pltpu.sync_copy(zeros_hbm, acc.at[pl.ds(s * ZROWS, ZROWS)])
    plsc.subcore_barrier()

    def grp_body(g, carry):
        @pl.when(g > 0)
        def _():
            wait_scatters(g - 1)

        @pl.when(g + 1 < ng)
        def _():
            wait_idx(g + 1)
            fire_gathers(g + 1)

        @pl.when(g + 3 < ng)
        def _():
            fire_idx(g + 3)

        wait_gathers(g)
        fire_scatters(g)
        return carry

    lax.fori_loop(0, ng, grp_body, 0)
    wait_scatters(ng - 1)

    @pl.when(wid == NW - 1)
    def _():
        pltpu.sync_copy(pads_hbm.at[1], sbuf.at[0])
        pltpu.sync_copy(pads_hbm.at[0], dbuf.at[0])
        for j in range(KB):
            pltpu.async_copy(x_hbm.at[sbuf.at[0, j]], rows.at[0, j],
                             gsem.at[0])
        for j in range(KB):
            pltpu.make_async_copy(
                x_hbm.at[sbuf.at[0, j]], rows.at[0, j], gsem.at[0]).wait()
        for j in range(KB):
            pltpu.sync_copy(rows.at[0, j], acc.at[dbuf.at[0, j]], add=True)

    plsc.subcore_barrier()

    pltpu.sync_copy(acc.at[pl.ds(s * ZROWS, ZROWS)],
                    part_hbm.at[c, pl.ds(s * ZROWS, ZROWS)])


def _make_hop(x_rows):
    mesh = plsc.VectorSubcoreMesh(core_axis_name="c", subcore_axis_name="s",
                                  num_cores=NC, num_subcores=NS)
    return pl.kernel(
        _hop_body,
        out_type=jax.ShapeDtypeStruct((NC, AR, C), jnp.float32),
        mesh=mesh,
        scratch_types=[
            pltpu.VMEM((4, KB, CHUNK), jnp.int32),
            pltpu.VMEM((4, KB, CHUNK), jnp.int32),
            pltpu.VMEM((2, KB, CHUNK, C), jnp.float32),
            pltpu.VMEM_SHARED((AR, C), jnp.float32),
            pltpu.SemaphoreType.DMA((4,)),
            pltpu.SemaphoreType.DMA((2,)),
            pltpu.SemaphoreType.DMA,
        ],
        compiler_params=pltpu.CompilerParams(use_tc_tiling_on_sc=False),
    )


_CROWS = 782
_CCHUNKS = AR // NW // _CROWS


def _comb_body(p_hbm, x_hbm, b0, b1, bo):
    c = lax.axis_index("c")
    s = lax.axis_index("s")
    wid = c * NS + s

    def chunk_body(ch, carry):
        base = wid * (AR // NW) + ch * _CROWS
        pltpu.sync_copy(p_hbm.at[0, pl.ds(base, _CROWS)], b0)
        pltpu.sync_copy(p_hbm.at[1, pl.ds(base, _CROWS)], b1)

        def row_body(i, carry2):
            bo[i, :] = b0[i, :] + b1[i, :]
            return carry2

        lax.fori_loop(0, _CROWS, row_body, 0)
        pltpu.sync_copy(bo, x_hbm.at[pl.ds(base, _CROWS)])
        return carry

    lax.fori_loop(0, _CCHUNKS, chunk_body, 0)


def _make_combine():
    mesh = plsc.VectorSubcoreMesh(core_axis_name="c", subcore_axis_name="s",
                                  num_cores=NC, num_subcores=NS)
    return pl.kernel(
        _comb_body,
        out_type=jax.ShapeDtypeStruct((AR, C), jnp.float32),
        mesh=mesh,
        scratch_types=[
            pltpu.VMEM((_CROWS, C), jnp.float32),
            pltpu.VMEM((_CROWS, C), jnp.float32),
            pltpu.VMEM((_CROWS, C), jnp.float32),
        ],
        compiler_params=pltpu.CompilerParams(use_tc_tiling_on_sc=False),
    )


_FROWS_IN = AR * C // 128
_FROWS = N * C // 128
_FB = 3128


def _final_body(p_ref, wbd_ref, sb_ref, bt_ref, o_ref):
    v = p_ref[0] + p_ref[1]
    z = jnp.dot(v, wbd_ref[...], preferred_element_type=jnp.float32)
    z = z + bt_ref[...]
    groups = []
    for k in range(8):
        mk = jnp.max(z[:, 16 * k:16 * (k + 1)], axis=1, keepdims=True)
        groups.append(jnp.broadcast_to(mk, (_FB, C)))
    m = jnp.concatenate(groups, axis=1)
    zz = z - m
    e = jnp.exp(zz)
    s = jnp.dot(e, sb_ref[...], preferred_element_type=jnp.float32)
    o_ref[...] = zz - jnp.log(s)


def _final(parts, wt, b2):
    pf = parts.reshape(NC, _FROWS_IN, 128)
    wbd = jnp.kron(jnp.eye(8, dtype=jnp.float32), wt)
    sb = jnp.kron(jnp.eye(8, dtype=jnp.float32),
                  jnp.ones((C, C), jnp.float32))
    bt = jnp.tile(b2, (1, 8))
    out = pl.pallas_call(
        _final_body,
        grid=(_FROWS_IN // _FB,),
        in_specs=[
            pl.BlockSpec((NC, _FB, 128), lambda i: (0, i, 0)),
            pl.BlockSpec((128, 128), lambda i: (0, 0)),
            pl.BlockSpec((128, 128), lambda i: (0, 0)),
            pl.BlockSpec((1, 128), lambda i: (0, 0)),
        ],
        out_specs=pl.BlockSpec((_FB, 128), lambda i: (i, 0)),
        out_shape=jax.ShapeDtypeStruct((_FROWS, 128), jnp.float32),
    )(pf, wbd, sb, bt)
    return out.reshape(N, C)


@jax.jit
def kernel(y, adj_t, W, b):
    adj3 = adj_t.astype(jnp.int32).reshape(2, TOTCH, CHUNK)
    npad = (KB - G_REM) * CHUNK
    tail = adj_t[:, G_FULL * KB * CHUNK:].astype(jnp.int32)
    pad_dst = N + (jnp.arange(npad, dtype=jnp.int32) % (AR - N))
    pad_src = jnp.zeros((npad,), jnp.int32)
    pads = jnp.concatenate(
        [tail, jnp.stack([pad_dst, pad_src])], axis=1).reshape(2, KB, CHUNK)
    zeros = jnp.zeros((ZROWS, C), jnp.float32)

    combine = _make_combine()
    p1 = _make_hop(N)(y, adj3, pads, zeros)
    x1 = combine(p1)
    p2 = _make_hop(AR)(x1, adj3, pads, zeros)
    return _final(p2, W.T, b.reshape(1, C))


# speedup vs baseline: 1.0594x; 1.0047x over previous
import jax
import jax.numpy as jnp
from jax import lax
from jax.experimental import pallas as pl
from jax.experimental.pallas import tpu as pltpu
from jax.experimental.pallas import tpu_sc as plsc

N = 100000
E = 3200000
C = 16

NC = 2
NS = 16
NW = NC * NS

CHUNK = 128
KB = 6
TOTCH = E // CHUNK
G_FULL = TOTCH // KB
G_REM = TOTCH - G_FULL * KB
BASE_G = G_FULL // NW
EXTRA = G_FULL % NW

AR = 100096
ZROWS = AR // NS


def _hop_body(x_hbm, adj3_hbm, pads_hbm, zeros_hbm, part_hbm,
              sbuf, dbuf, rows, acc, isem, gsem, ssem):
    c = lax.axis_index("c")
    s = lax.axis_index("s")
    wid = c * NS + s
    g0 = BASE_G * wid + jnp.minimum(wid, EXTRA)
    ng = BASE_G + jnp.where(wid < EXTRA, 1, 0)

    def fire_idx(g):
        r = g % 4
        q = (g0 + g) * KB
        pltpu.async_copy(adj3_hbm.at[1, pl.ds(q, KB)], sbuf.at[r], isem.at[r])
        pltpu.async_copy(adj3_hbm.at[0, pl.ds(q, KB)], dbuf.at[r], isem.at[r])

    def wait_idx(g):
        r = g % 4
        q = (g0 + g) * KB
        pltpu.make_async_copy(
            adj3_hbm.at[1, pl.ds(q, KB)], sbuf.at[r], isem.at[r]).wait()
        pltpu.make_async_copy(
            adj3_hbm.at[0, pl.ds(q, KB)], dbuf.at[r], isem.at[r]).wait()

    def fire_gathers(g):
        r, p = g % 4, g % 2
        for j in range(KB):
            pltpu.async_copy(x_hbm.at[sbuf.at[r, j]], rows.at[p, j],
                             gsem.at[p])

    def wait_gathers(g):
        r, p = g % 4, g % 2
        for j in range(KB):
            pltpu.make_async_copy(
                x_hbm.at[sbuf.at[r, j]], rows.at[p, j], gsem.at[p]).wait()

    def fire_scatters(g):
        r, p = g % 4, g % 2
        for j in range(KB):
            pltpu.async_copy(rows.at[p, j], acc.at[dbuf.at[r, j]], ssem,
                             add=True)

    def wait_scatters(g):
        r, p = g % 4, g % 2
        for j in range(KB):
            pltpu.make_async_copy(
                rows.at[p, j], acc.at[dbuf.at[r, j]], ssem).wait()

    fire_idx(0)
    fire_idx(1)
    fire_idx(2)
    wait_idx(0)
    fire_gathers(0)
    pltpu.sync_copy(zeros_hbm, acc.at[pl.ds(s * ZROWS, ZROWS)])
    plsc.subcore_barrier()

    def grp_body(g, carry):
        @pl.when(g > 0)
        def _():
            wait_scatters(g - 1)

        @pl.when(g + 1 < ng)
        def _():
            wait_idx(g + 1)
            fire_gathers(g + 1)

        @pl.when(g + 3 < ng)
        def _():
            fire_idx(g + 3)

        wait_gathers(g)
        fire_scatters(g)
        return carry

    lax.fori_loop(0, ng, grp_body, 0)
    wait_scatters(ng - 1)

    @pl.when(wid == NW - 1)
    def _():
        pltpu.sync_copy(pads_hbm.at[1], sbuf.at[0])
        pltpu.sync_copy(pads_hbm.at[0], dbuf.at[0])
        for j in range(KB):
            pltpu.async_copy(x_hbm.at[sbuf.at[0, j]], rows.at[0, j],
                             gsem.at[0])
        for j in range(KB):
            pltpu.make_async_copy(
                x_hbm.at[sbuf.at[0, j]], rows.at[0, j], gsem.at[0]).wait()
        for j in range(KB):
            pltpu.sync_copy(rows.at[0, j], acc.at[dbuf.at[0, j]], add=True)

    plsc.subcore_barrier()

    pltpu.sync_copy(acc.at[pl.ds(s * ZROWS, ZROWS)],
                    part_hbm.at[c, pl.ds(s * ZROWS, ZROWS)])


def _make_hop(x_rows):
    mesh = plsc.VectorSubcoreMesh(core_axis_name="c", subcore_axis_name="s",
                                  num_cores=NC, num_subcores=NS)
    return pl.kernel(
        _hop_body,
        out_type=jax.ShapeDtypeStruct((NC, AR, C), jnp.float32),
        mesh=mesh,
        scratch_types=[
            pltpu.VMEM((4, KB, CHUNK), jnp.int32),
            pltpu.VMEM((4, KB, CHUNK), jnp.int32),
            pltpu.VMEM((2, KB, CHUNK, C), jnp.float32),
            pltpu.VMEM_SHARED((AR, C), jnp.float32),
            pltpu.SemaphoreType.DMA((4,)),
            pltpu.SemaphoreType.DMA((2,)),
            pltpu.SemaphoreType.DMA,
        ],
        compiler_params=pltpu.CompilerParams(use_tc_tiling_on_sc=False),
    )


_CROWS = 782
_CCHUNKS = AR // NW // _CROWS


def _comb_body(p_hbm, x_hbm, b0, b1, bo, lsem, osem):
    c = lax.axis_index("c")
    s = lax.axis_index("s")
    wid = c * NS + s
    w0 = wid * (AR // NW)

    def fire_loads(ch):
        p = ch % 2
        base = w0 + ch * _CROWS
        pltpu.async_copy(p_hbm.at[0, pl.ds(base, _CROWS)], b0.at[p],
                         lsem.at[p])
        pltpu.async_copy(p_hbm.at[1, pl.ds(base, _CROWS)], b1.at[p],
                         lsem.at[p])

    def wait_loads(ch):
        p = ch % 2
        base = w0 + ch * _CROWS
        pltpu.make_async_copy(
            p_hbm.at[0, pl.ds(base, _CROWS)], b0.at[p], lsem.at[p]).wait()
        pltpu.make_async_copy(
            p_hbm.at[1, pl.ds(base, _CROWS)], b1.at[p], lsem.at[p]).wait()

    def fire_store(ch):
        p = ch % 2
        base = w0 + ch * _CROWS
        pltpu.async_copy(bo.at[p], x_hbm.at[pl.ds(base, _CROWS)], osem)

    def wait_store(ch):
        p = ch % 2
        base = w0 + ch * _CROWS
        pltpu.make_async_copy(
            bo.at[p], x_hbm.at[pl.ds(base, _CROWS)], osem).wait()

    fire_loads(0)

    def chunk_body(ch, carry):
        p = ch % 2

        @pl.when(ch + 1 < _CCHUNKS)
        def _():
            fire_loads(ch + 1)

        wait_loads(ch)

        @pl.when(ch > 1)
        def _():
            wait_store(ch - 2)

        def row_body(i, carry2):
            bo[p, i, :] = b0[p, i, :] + b1[p, i, :]
            return carry2

        lax.fori_loop(0, _CROWS, row_body, 0)
        fire_store(ch)
        return carry

    lax.fori_loop(0, _CCHUNKS, chunk_body, 0)
    wait_store(_CCHUNKS - 2)
    wait_store(_CCHUNKS - 1)


def _make_combine():
    mesh = plsc.VectorSubcoreMesh(core_axis_name="c", subcore_axis_name="s",
                                  num_cores=NC, num_subcores=NS)
    return pl.kernel(
        _comb_body,
        out_type=jax.ShapeDtypeStruct((AR, C), jnp.float32),
        mesh=mesh,
        scratch_types=[
            pltpu.VMEM((2, _CROWS, C), jnp.float32),
            pltpu.VMEM((2, _CROWS, C), jnp.float32),
            pltpu.VMEM((2, _CROWS, C), jnp.float32),
            pltpu.SemaphoreType.DMA((2,)),
            pltpu.SemaphoreType.DMA,
        ],
        compiler_params=pltpu.CompilerParams(use_tc_tiling_on_sc=False),
    )


_FROWS_IN = AR * C // 128
_FROWS = N * C // 128
_FB = 3128


def _final_body(p_ref, wbd_ref, sb_ref, bt_ref, o_ref):
    v = p_ref[0] + p_ref[1]
    z = jnp.dot(v, wbd_ref[...], preferred_element_type=jnp.float32)
    z = z + bt_ref[...]
    groups = []
    for k in range(8):
        mk = jnp.max(z[:, 16 * k:16 * (k + 1)], axis=1, keepdims=True)
        groups.append(jnp.broadcast_to(mk, (_FB, C)))
    m = jnp.concatenate(groups, axis=1)
    zz = z - m
    e = jnp.exp(zz)
    s = jnp.dot(e, sb_ref[...], preferred_element_type=jnp.float32)
    o_ref[...] = zz - jnp.log(s)


def _final(parts, wt, b2):
    pf = parts.reshape(NC, _FROWS_IN, 128)
    wbd = jnp.kron(jnp.eye(8, dtype=jnp.float32), wt)
    sb = jnp.kron(jnp.eye(8, dtype=jnp.float32),
                  jnp.ones((C, C), jnp.float32))
    bt = jnp.tile(b2, (1, 8))
    out = pl.pallas_call(
        _final_body,
        grid=(_FROWS_IN // _FB,),
        in_specs=[
            pl.BlockSpec((NC, _FB, 128), lambda i: (0, i, 0)),
            pl.BlockSpec((128, 128), lambda i: (0, 0)),
            pl.BlockSpec((128, 128), lambda i: (0, 0)),
            pl.BlockSpec((1, 128), lambda i: (0, 0)),
        ],
        out_specs=pl.BlockSpec((_FB, 128), lambda i: (i, 0)),
        out_shape=jax.ShapeDtypeStruct((_FROWS, 128), jnp.float32),
    )(pf, wbd, sb, bt)
    return out.reshape(N, C)


@jax.jit
def kernel(y, adj_t, W, b):
    adj3 = adj_t.astype(jnp.int32).reshape(2, TOTCH, CHUNK)
    npad = (KB - G_REM) * CHUNK
    tail = adj_t[:, G_FULL * KB * CHUNK:].astype(jnp.int32)
    pad_dst = N + (jnp.arange(npad, dtype=jnp.int32) % (AR - N))
    pad_src = jnp.zeros((npad,), jnp.int32)
    pads = jnp.concatenate(
        [tail, jnp.stack([pad_dst, pad_src])], axis=1).reshape(2, KB, CHUNK)
    zeros = jnp.zeros((ZROWS, C), jnp.float32)

    combine = _make_combine()
    p1 = _make_hop(N)(y, adj3, pads, zeros)
    x1 = combine(p1)
    p2 = _make_hop(AR)(x1, adj3, pads, zeros)
    return _final(p2, W.T, b.reshape(1, C))


# speedup vs baseline: 1.0883x; 1.0273x over previous
import jax
import jax.numpy as jnp
from jax import lax
from jax.experimental import pallas as pl
from jax.experimental.pallas import tpu as pltpu
from jax.experimental.pallas import tpu_sc as plsc

N = 100000
E = 3200000
C = 16

NC = 2
NS = 16
NW = NC * NS

CHUNK = 128
KB = 6
TOTCH = E // CHUNK
G_FULL = TOTCH // KB
G_REM = TOTCH - G_FULL * KB
BASE_G = G_FULL // NW
EXTRA = G_FULL % NW

AR = 100096
ZROWS = AR // NS


def _hop_body(x_hbm, adj3_hbm, pads_hbm, zeros_hbm, part_hbm,
              sbuf, dbuf, rows, acc, isem, gsem, ssem):
    c = lax.axis_index("c")
    s = lax.axis_index("s")
    wid = c * NS + s
    g0 = BASE_G * wid + jnp.minimum(wid, EXTRA)
    ng = BASE_G + jnp.where(wid < EXTRA, 1, 0)

    def fire_idx(g):
        r = g % 4
        q = (g0 + g) * KB
        pltpu.async_copy(adj3_hbm.at[1, pl.ds(q, KB)], sbuf.at[r], isem.at[r])
        pltpu.async_copy(adj3_hbm.at[0, pl.ds(q, KB)], dbuf.at[r], isem.at[r])

    def wait_idx(g):
        r = g % 4
        q = (g0 + g) * KB
        pltpu.make_async_copy(
            adj3_hbm.at[1, pl.ds(q, KB)], sbuf.at[r], isem.at[r]).wait()
        pltpu.make_async_copy(
            adj3_hbm.at[0, pl.ds(q, KB)], dbuf.at[r], isem.at[r]).wait()

    def fire_gathers(g):
        r, p = g % 4, g % 2
        for j in range(KB):
            pltpu.async_copy(x_hbm.at[sbuf.at[r, j]], rows.at[p, j],
                             gsem.at[p])

    def wait_gathers(g):
        r, p = g % 4, g % 2
        for j in range(KB):
            pltpu.make_async_copy(
                x_hbm.at[sbuf.at[r, j]], rows.at[p, j], gsem.at[p]).wait()

    def fire_scatters(g):
        r, p = g % 4, g % 2
        for j in range(KB):
            pltpu.async_copy(rows.at[p, j], acc.at[dbuf.at[r, j]], ssem,
                             add=True)

    def wait_scatters(g):
        r, p = g % 4, g % 2
        for j in range(KB):
            pltpu.make_async_copy(
                rows.at[p, j], acc.at[dbuf.at[r, j]], ssem).wait()

    fire_idx(0)
    fire_idx(1)
    fire_idx(2)
    wait_idx(0)
    fire_gathers(0)
    pltpu.sync_copy(zeros_hbm, acc.at[pl.ds(s * ZROWS, ZROWS)])
    plsc.subcore_barrier()

    def grp_body(g, carry):
        @pl.when(g > 0)
        def _():
            wait_scatters(g - 1)

        @pl.when(g + 1 < ng)
        def _():
            wait_idx(g + 1)
            fire_gathers(g + 1)

        @pl.when(g + 3 < ng)
        def _():
            fire_idx(g + 3)

        wait_gathers(g)
        fire_scatters(g)
        return carry

    lax.fori_loop(0, ng, grp_body, 0)
    wait_scatters(ng - 1)

    @pl.when(wid == NW - 1)
    def _():
        pltpu.sync_copy(pads_hbm.at[1], sbuf.at[0])
        pltpu.sync_copy(pads_hbm.at[0], dbuf.at[0])
        for j in range(KB):
            pltpu.async_copy(x_hbm.at[sbuf.at[0, j]], rows.at[0, j],
                             gsem.at[0])
        for j in range(KB):
            pltpu.make_async_copy(
                x_hbm.at[sbuf.at[0, j]], rows.at[0, j], gsem.at[0]).wait()
        for j in range(KB):
            pltpu.sync_copy(rows.at[0, j], acc.at[dbuf.at[0, j]], add=True)

    plsc.subcore_barrier()

    pltpu.sync_copy(acc.at[pl.ds(s * ZROWS, ZROWS)],
                    part_hbm.at[c, pl.ds(s * ZROWS, ZROWS)])


def _make_hop(x_rows):
    mesh = plsc.VectorSubcoreMesh(core_axis_name="c", subcore_axis_name="s",
                                  num_cores=NC, num_subcores=NS)
    return pl.kernel(
        _hop_body,
        out_type=jax.ShapeDtypeStruct((NC, AR, C), jnp.float32),
        mesh=mesh,
        scratch_types=[
            pltpu.VMEM((4, KB, CHUNK), jnp.int32),
            pltpu.VMEM((4, KB, CHUNK), jnp.int32),
            pltpu.VMEM((2, KB, CHUNK, C), jnp.float32),
            pltpu.VMEM_SHARED((AR, C), jnp.float32),
            pltpu.SemaphoreType.DMA((4,)),
            pltpu.SemaphoreType.DMA((2,)),
            pltpu.SemaphoreType.DMA,
        ],
        compiler_params=pltpu.CompilerParams(use_tc_tiling_on_sc=False),
    )


_CROWS = 782
_CCHUNKS = AR // NW // _CROWS


def _comb_body(p_hbm, x_hbm, b0, b1, bo, lsem, osem):
    c = lax.axis_index("c")
    s = lax.axis_index("s")
    wid = c * NS + s
    w0 = wid * (AR // NW)

    def fire_loads(ch):
        p = ch % 2
        base = w0 + ch * _CROWS
        pltpu.async_copy(p_hbm.at[0, pl.ds(base, _CROWS)], b0.at[p],
                         lsem.at[p])
        pltpu.async_copy(p_hbm.at[1, pl.ds(base, _CROWS)], b1.at[p],
                         lsem.at[p])

    def wait_loads(ch):
        p = ch % 2
        base = w0 + ch * _CROWS
        pltpu.make_async_copy(
            p_hbm.at[0, pl.ds(base, _CROWS)], b0.at[p], lsem.at[p]).wait()
        pltpu.make_async_copy(
            p_hbm.at[1, pl.ds(base, _CROWS)], b1.at[p], lsem.at[p]).wait()

    def fire_store(ch):
        p = ch % 2
        base = w0 + ch * _CROWS
        pltpu.async_copy(bo.at[p], x_hbm.at[pl.ds(base, _CROWS)], osem)

    def wait_store(ch):
        p = ch % 2
        base = w0 + ch * _CROWS
        pltpu.make_async_copy(
            bo.at[p], x_hbm.at[pl.ds(base, _CROWS)], osem).wait()

    fire_loads(0)

    def chunk_body(ch, carry):
        p = ch % 2

        @pl.when(ch + 1 < _CCHUNKS)
        def _():
            fire_loads(ch + 1)

        wait_loads(ch)

        @pl.when(ch > 1)
        def _():
            wait_store(ch - 2)

        @plsc.parallel_loop(0, _CROWS, unroll=4)
        def row_body(i):
            bo[p, i, :] = b0[p, i, :] + b1[p, i, :]

        fire_store(ch)
        return carry

    lax.fori_loop(0, _CCHUNKS, chunk_body, 0)
    wait_store(_CCHUNKS - 2)
    wait_store(_CCHUNKS - 1)


def _make_combine():
    mesh = plsc.VectorSubcoreMesh(core_axis_name="c", subcore_axis_name="s",
                                  num_cores=NC, num_subcores=NS)
    return pl.kernel(
        _comb_body,
        out_type=jax.ShapeDtypeStruct((AR, C), jnp.float32),
        mesh=mesh,
        scratch_types=[
            pltpu.VMEM((2, _CROWS, C), jnp.float32),
            pltpu.VMEM((2, _CROWS, C), jnp.float32),
            pltpu.VMEM((2, _CROWS, C), jnp.float32),
            pltpu.SemaphoreType.DMA((2,)),
            pltpu.SemaphoreType.DMA,
        ],
        compiler_params=pltpu.CompilerParams(use_tc_tiling_on_sc=False),
    )


_FROWS_IN = AR * C // 128
_FROWS = N * C // 128
_FB = 3128


def _final_body(p_ref, wbd_ref, sb_ref, bt_ref, o_ref):
    v = p_ref[0] + p_ref[1]
    z = jnp.dot(v, wbd_ref[...], preferred_element_type=jnp.float32)
    z = z + bt_ref[...]
    groups = []
    for k in range(8):
        mk = jnp.max(z[:, 16 * k:16 * (k + 1)], axis=1, keepdims=True)
        groups.append(jnp.broadcast_to(mk, (_FB, C)))
    m = jnp.concatenate(groups, axis=1)
    zz = z - m
    e = jnp.exp(zz)
    s = jnp.dot(e, sb_ref[...], preferred_element_type=jnp.float32)
    o_ref[...] = zz - jnp.log(s)


def _final(parts, wt, b2):
    pf = parts.reshape(NC, _FROWS_IN, 128)
    wbd = jnp.kron(jnp.eye(8, dtype=jnp.float32), wt)
    sb = jnp.kron(jnp.eye(8, dtype=jnp.float32),
                  jnp.ones((C, C), jnp.float32))
    bt = jnp.tile(b2, (1, 8))
    out = pl.pallas_call(
        _final_body,
        grid=(_FROWS_IN // _FB,),
        in_specs=[
            pl.BlockSpec((NC, _FB, 128), lambda i: (0, i, 0)),
            pl.BlockSpec((128, 128), lambda i: (0, 0)),
            pl.BlockSpec((128, 128), lambda i: (0, 0)),
            pl.BlockSpec((1, 128), lambda i: (0, 0)),
        ],
        out_specs=pl.BlockSpec((_FB, 128), lambda i: (i, 0)),
        out_shape=jax.ShapeDtypeStruct((_FROWS, 128), jnp.float32),
    )(pf, wbd, sb, bt)
    return out.reshape(N, C)


@jax.jit
def kernel(y, adj_t, W, b):
    adj3 = adj_t.astype(jnp.int32).reshape(2, TOTCH, CHUNK)
    npad = (KB - G_REM) * CHUNK
    tail = adj_t[:, G_FULL * KB * CHUNK:].astype(jnp.int32)
    pad_dst = N + (jnp.arange(npad, dtype=jnp.int32) % (AR - N))
    pad_src = jnp.zeros((npad,), jnp.int32)
    pads = jnp.concatenate(
        [tail, jnp.stack([pad_dst, pad_src])], axis=1).reshape(2, KB, CHUNK)
    zeros = jnp.zeros((ZROWS, C), jnp.float32)

    combine = _make_combine()
    p1 = _make_hop(N)(y, adj3, pads, zeros)
    x1 = combine(p1)
    p2 = _make_hop(AR)(x1, adj3, pads, zeros)
    return _final(p2, W.T, b.reshape(1, C))
